# baseline (speedup 1.0000x reference)
import jax
import jax.numpy as jnp
from jax import lax
from jax.experimental import pallas as pl
from jax.experimental.pallas import tpu as pltpu
from jax.experimental.pallas import tpu_sc as plsc

NC = 2
NS = 16
L = 16
NW = NC * NS

B = 16384
D = 32
N_EX = 100000
BPW = B // NW
CH = 128
NCHUNK = BPW // CH
GPC = CH // L


def _mirt_body(user_hbm, item_hbm, theta_hbm, a_hbm, b_hbm, out_hbm, *scr):
    idx_u = scr[0:NCHUNK]
    idx_i = scr[NCHUNK:2 * NCHUNK]
    idx_b = scr[2 * NCHUNK:3 * NCHUNK]
    th_v = scr[3 * NCHUNK:4 * NCHUNK]
    a_v = scr[4 * NCHUNK:5 * NCHUNK]
    b_v = scr[5 * NCHUNK:6 * NCHUNK]
    out_v = scr[6 * NCHUNK]
    sem_t = scr[6 * NCHUNK + 1:7 * NCHUNK + 1]
    sem_a = scr[7 * NCHUNK + 1:8 * NCHUNK + 1]
    sem_b = scr[8 * NCHUNK + 1:9 * NCHUNK + 1]

    wid = lax.axis_index("s") * NC + lax.axis_index("c")
    base = wid * BPW

    for c in range(NCHUNK):
        pltpu.sync_copy(user_hbm.at[pl.ds(base + c * CH, CH)], idx_u[c])
        pltpu.sync_copy(item_hbm.at[pl.ds(base + c * CH, CH)], idx_i[c])
    for c in range(NCHUNK):
        for k in range(GPC):
            idx_b[c][pl.ds(k * L, L)] = idx_i[c][pl.ds(k * L, L)] >> 4
    copies = []
    for c in range(NCHUNK):
        ct = pltpu.async_copy(theta_hbm.at[idx_u[c]], th_v[c], sem_t[c])
        ca = pltpu.async_copy(a_hbm.at[idx_i[c]], a_v[c], sem_a[c])
        cb = pltpu.async_copy(b_hbm.at[idx_b[c]], b_v[c], sem_b[c])
        copies.append((ct, ca, cb))

    lane = lax.iota(jnp.int32, L)

    for c in range(NCHUNK):
        ct, ca, cb = copies[c]
        ct.wait()
        ca.wait()
        cb.wait()

        def group_body(g, _, c=c):
            rows = g * L + lane
            acc = jnp.zeros((L,), jnp.float32)
            for j in range(D):
                jj = jnp.full((L,), j, jnp.int32)
                t = plsc.load_gather(th_v[c], [rows, jj])
                av = plsc.load_gather(a_v[c], [rows, jj])
                acc = acc + (t + t) / (1.0 + jnp.exp(-av))
            itv = idx_i[c][pl.ds(g * L, L)]
            bv = plsc.load_gather(b_v[c], [rows, itv & 15])
            res = 1.0 / (1.0 + jnp.exp(bv - acc))
            out_v[pl.ds(c * CH + g * L, L)] = res
            return 0

        lax.fori_loop(0, GPC, group_body, 0)

    pltpu.sync_copy(out_v, out_hbm.at[pl.ds(base, BPW)])


@jax.jit
def kernel(user, item, theta_table, a_table, b_table):
    b_blocks = b_table.reshape(N_EX // L, L)
    mesh = plsc.VectorSubcoreMesh(
        core_axis_name="c", subcore_axis_name="s",
        num_cores=NC, num_subcores=NS)
    scratch = (
        [pltpu.VMEM((CH,), jnp.int32) for _ in range(NCHUNK)]
        + [pltpu.VMEM((CH,), jnp.int32) for _ in range(NCHUNK)]
        + [pltpu.VMEM((CH,), jnp.int32) for _ in range(NCHUNK)]
        + [pltpu.VMEM((CH, D), jnp.float32) for _ in range(NCHUNK)]
        + [pltpu.VMEM((CH, D), jnp.float32) for _ in range(NCHUNK)]
        + [pltpu.VMEM((CH, L), jnp.float32) for _ in range(NCHUNK)]
        + [pltpu.VMEM((BPW,), jnp.float32)]
        + [pltpu.SemaphoreType.DMA for _ in range(3 * NCHUNK)]
    )
    f = pl.kernel(
        _mirt_body,
        out_type=jax.ShapeDtypeStruct((B,), jnp.float32),
        mesh=mesh,
        compiler_params=pltpu.CompilerParams(
            needs_layout_passes=False, use_tc_tiling_on_sc=False),
        scratch_types=scratch,
    )
    return f(user, item, theta_table, a_table, b_blocks)
